# baseline (speedup 1.0000x reference)
import jax
import jax.numpy as jnp
from jax.experimental import pallas as pl
from jax.experimental.pallas import tpu as pltpu
from jax.experimental.pallas import tpu_sc as plsc

_EMBED = 16
_LANES = 128
_NUM_WORKERS = 32
_GROUP = 128
_BLK = 4096


def _sc_gather(table, idx):
    n = idx.shape[0]
    per = n // _NUM_WORKERS
    num_groups = per // _GROUP
    mesh = plsc.VectorSubcoreMesh(core_axis_name="core", subcore_axis_name="subcore")

    @pl.kernel(
        out_type=jax.ShapeDtypeStruct((n, _LANES), table.dtype),
        mesh=mesh,
        scratch_types=[
            pltpu.VMEM((per,), jnp.int32),
            pltpu.VMEM((_GROUP, _LANES), table.dtype),
            pltpu.SemaphoreType.DMA,
            pltpu.SemaphoreType.DMA,
        ],
    )
    def gather_kernel(x_hbm, i_hbm, o_hbm, idx_vmem, buf, sem_idx, sem_row):
        core = jax.lax.axis_index("core")
        sub = jax.lax.axis_index("subcore")
        base = (core * 16 + sub) * per
        pltpu.async_copy(i_hbm.at[pl.ds(base, per)], idx_vmem, sem_idx).wait()

        for g in range(num_groups):

            @pl.loop(0, _GROUP // 16)
            def _issue(c):
                v = idx_vmem[pl.ds(g * _GROUP + c * 16, 16)]
                for k in range(16):
                    pltpu.async_copy(
                        x_hbm.at[v[k]],
                        buf.at[c * 16 + k, pl.ds(0, _EMBED)],
                        sem_row,
                    )

            @pl.loop(0, _GROUP)
            def _drain(j):
                pltpu.make_async_copy(
                    x_hbm.at[0], buf.at[j, pl.ds(0, _EMBED)], sem_row
                ).wait()

            pltpu.async_copy(
                buf, o_hbm.at[pl.ds(base + g * _GROUP, _GROUP), :], sem_idx
            ).wait()

    return gather_kernel(table, idx)


def _tc_mlp(gh, ga, W1a, W1b, b1, W2, b2, W3, b3):
    batch = gh.shape[0]

    def body(gh_ref, ga_ref, w1a_ref, w1b_ref, b1_ref, w2_ref, b2_ref, w3_ref,
             b3_ref, o_ref):
        eh = gh_ref[:, : _EMBED]
        ea = ga_ref[:, : _EMBED]
        d = jnp.abs(eh - ea)
        p = eh * ea
        h = (
            jnp.dot(d, w1a_ref[...], preferred_element_type=jnp.float32)
            + jnp.dot(p, w1b_ref[...], preferred_element_type=jnp.float32)
            + b1_ref[...]
        )
        h = jnp.maximum(h, 0.0)
        h = jnp.dot(h, w2_ref[...], preferred_element_type=jnp.float32) + b2_ref[...]
        h = jnp.maximum(h, 0.0)
        o_ref[...] = (
            jnp.dot(h, w3_ref[...], preferred_element_type=jnp.float32) + b3_ref[...]
        )

    grid = (batch // _BLK,)
    row_spec = lambda w: pl.BlockSpec((_BLK, w), lambda i: (i, 0))
    full = lambda a: pl.BlockSpec(a.shape, lambda i: (0,) * a.ndim)
    return pl.pallas_call(
        body,
        grid=grid,
        in_specs=[
            row_spec(_LANES), row_spec(_LANES),
            full(W1a), full(W1b), full(b1), full(W2), full(b2), full(W3), full(b3),
        ],
        out_specs=pl.BlockSpec((_BLK, 3), lambda i: (i, 0)),
        out_shape=jax.ShapeDtypeStruct((batch, 3), jnp.float32),
    )(gh, ga, W1a, W1b, b1, W2, b2, W3, b3)


def kernel(home_ids, away_ids, table, W1, b1, W2, b2, W3, b3):
    batch = home_ids.shape[0]
    ids = jnp.concatenate([home_ids, away_ids], axis=0).astype(jnp.int32)
    g = _sc_gather(table, ids)
    return _tc_mlp(
        g[:batch], g[batch:],
        W1[:_EMBED], W1[_EMBED:],
        b1.reshape(1, -1), W2, b2.reshape(1, -1), W3, b3.reshape(1, -1),
    )


# speedup vs baseline: 1.0051x; 1.0051x over previous
import jax
import jax.numpy as jnp
from jax.experimental import pallas as pl
from jax.experimental.pallas import tpu as pltpu
from jax.experimental.pallas import tpu_sc as plsc

_EMBED = 16
_LANES = 128
_NUM_WORKERS = 32
_GROUP = 128
_BLK = 4096


def _sc_gather(table, idx):
    n = idx.shape[0]
    per = n // _NUM_WORKERS
    num_groups = per // _GROUP
    mesh = plsc.VectorSubcoreMesh(core_axis_name="core", subcore_axis_name="subcore")

    @pl.kernel(
        out_type=jax.ShapeDtypeStruct((n, _LANES), table.dtype),
        mesh=mesh,
        scratch_types=[
            pltpu.VMEM((per,), jnp.int32),
            pltpu.VMEM((_GROUP, _LANES), table.dtype),
            pltpu.SemaphoreType.DMA,
            pltpu.SemaphoreType.DMA,
        ],
    )
    def gather_kernel(x_hbm, i_hbm, o_hbm, idx_vmem, buf, sem_idx, sem_row):
        core = jax.lax.axis_index("core")
        sub = jax.lax.axis_index("subcore")
        base = (core * 16 + sub) * per
        pltpu.async_copy(i_hbm.at[pl.ds(base, per)], idx_vmem, sem_idx).wait()

        for g in range(num_groups):

            @plsc.parallel_loop(0, _GROUP // 16)
            def _issue(c):
                v = idx_vmem[pl.ds(g * _GROUP + c * 16, 16)]
                for k in range(16):
                    pltpu.async_copy(
                        x_hbm.at[v[k]],
                        buf.at[c * 16 + k, pl.ds(0, _EMBED)],
                        sem_row,
                    )

            @pl.loop(0, _GROUP)
            def _drain(j):
                pltpu.make_async_copy(
                    x_hbm.at[0], buf.at[j, pl.ds(0, _EMBED)], sem_row
                ).wait()

            pltpu.async_copy(
                buf, o_hbm.at[pl.ds(base + g * _GROUP, _GROUP), :], sem_idx
            ).wait()

    return gather_kernel(table, idx)


def _tc_mlp(gh, ga, W1a, W1b, b1, W2, b2, W3, b3):
    batch = gh.shape[0]

    def body(gh_ref, ga_ref, w1a_ref, w1b_ref, b1_ref, w2_ref, b2_ref, w3_ref,
             b3_ref, o_ref):
        eh = gh_ref[:, : _EMBED]
        ea = ga_ref[:, : _EMBED]
        d = jnp.abs(eh - ea)
        p = eh * ea
        h = (
            jnp.dot(d, w1a_ref[...], preferred_element_type=jnp.float32)
            + jnp.dot(p, w1b_ref[...], preferred_element_type=jnp.float32)
            + b1_ref[...]
        )
        h = jnp.maximum(h, 0.0)
        h = jnp.dot(h, w2_ref[...], preferred_element_type=jnp.float32) + b2_ref[...]
        h = jnp.maximum(h, 0.0)
        o_ref[...] = (
            jnp.dot(h, w3_ref[...], preferred_element_type=jnp.float32) + b3_ref[...]
        )

    grid = (batch // _BLK,)
    row_spec = lambda w: pl.BlockSpec((_BLK, w), lambda i: (i, 0))
    full = lambda a: pl.BlockSpec(a.shape, lambda i: (0,) * a.ndim)
    return pl.pallas_call(
        body,
        grid=grid,
        in_specs=[
            row_spec(_LANES), row_spec(_LANES),
            full(W1a), full(W1b), full(b1), full(W2), full(b2), full(W3), full(b3),
        ],
        out_specs=pl.BlockSpec((_BLK, 3), lambda i: (i, 0)),
        out_shape=jax.ShapeDtypeStruct((batch, 3), jnp.float32),
    )(gh, ga, W1a, W1b, b1, W2, b2, W3, b3)


def kernel(home_ids, away_ids, table, W1, b1, W2, b2, W3, b3):
    batch = home_ids.shape[0]
    ids = jnp.concatenate([home_ids, away_ids], axis=0).astype(jnp.int32)
    g = _sc_gather(table, ids)
    return _tc_mlp(
        g[:batch], g[batch:],
        W1[:_EMBED], W1[_EMBED:],
        b1.reshape(1, -1), W2, b2.reshape(1, -1), W3, b3.reshape(1, -1),
    )
